# baseline (speedup 1.0000x reference)
import functools

import jax
import jax.numpy as jnp
from jax import lax
from jax.experimental import pallas as pl
from jax.experimental.pallas import tpu as pltpu
from jax.experimental.pallas import tpu_sc as plsc

_EPS = 0.1
_N, _C, _W = 8192, 32000, 5
_NC, _NS, _L = 2, 16, 16
_NW = _NC * _NS
_RPW = _N // _NW
_TGT = _RPW
_WIN = _RPW * _W
_TOT = _TGT + _WIN
_CH = 128
_NCH = _TOT // _CH


def _loss_body(pred_hbm, tgt_hbm, smt_hbm, out_hbm,
               tgt_v, sm_v, idx_v, val_v, part_v, sem):
    cid = lax.axis_index("c")
    sid = lax.axis_index("s")
    wid = sid * _NC + cid
    base = wid * _RPW

    pltpu.sync_copy(tgt_hbm.at[pl.ds(base, _RPW)], tgt_v)
    for w in range(_W):
        pltpu.sync_copy(smt_hbm.at[pl.ds(w * _N + base, _RPW)],
                        sm_v.at[pl.ds(w * _RPW, _RPW)])

    lanes = lax.iota(jnp.int32, _L)
    for k in range(_RPW // _L):
        rows = base + k * _L + lanes
        idx_v[pl.ds(k * _L, _L)] = rows * _C + tgt_v[pl.ds(k * _L, _L)]
    for w in range(_W):
        for k in range(_RPW // _L):
            rows = base + k * _L + lanes
            off = _TGT + w * _RPW + k * _L
            idx_v[pl.ds(off, _L)] = rows * _C + sm_v[pl.ds(w * _RPW + k * _L, _L)]

    copies = [
        pltpu.async_copy(pred_hbm.at[idx_v.at[pl.ds(c * _CH, _CH)]],
                         val_v.at[pl.ds(c * _CH, _CH)], sem)
        for c in range(_NCH)
    ]
    for cp in copies:
        cp.wait()

    acc_t = jnp.zeros((_L,), jnp.float32)
    for k in range(_TGT // _L):
        acc_t = acc_t + val_v[pl.ds(k * _L, _L)]
    acc_w = jnp.zeros((_L,), jnp.float32)
    for k in range(_WIN // _L):
        acc_w = acc_w + val_v[pl.ds(_TGT + k * _L, _L)]
    part_v[...] = -(acc_t * ((1.0 - _EPS) / _N) + acc_w * (_EPS / (_N * _W)))
    pltpu.sync_copy(part_v, out_hbm.at[wid])


@functools.partial(
    pl.kernel,
    out_type=jax.ShapeDtypeStruct((_NW, _L), jnp.float32),
    mesh=plsc.VectorSubcoreMesh(core_axis_name="c", subcore_axis_name="s"),
    scratch_types=[
        pltpu.VMEM((_TGT,), jnp.int32),
        pltpu.VMEM((_WIN,), jnp.int32),
        pltpu.VMEM((_TOT,), jnp.int32),
        pltpu.VMEM((_TOT,), jnp.float32),
        pltpu.VMEM((_L,), jnp.float32),
        pltpu.SemaphoreType.DMA,
    ],
)
def _sc_loss(pred_hbm, tgt_hbm, smt_hbm, out_hbm,
             tgt_v, sm_v, idx_v, val_v, part_v, sem):
    _loss_body(pred_hbm, tgt_hbm, smt_hbm, out_hbm,
               tgt_v, sm_v, idx_v, val_v, part_v, sem)


def kernel(pred, target, smooth_idx):
    pred_flat = pred.reshape(-1)
    tgt = target.astype(jnp.int32)
    smt = smooth_idx.astype(jnp.int32).T.reshape(-1)
    parts = _sc_loss(pred_flat, tgt, smt)
    return jnp.sum(parts)


# speedup vs baseline: 27.0622x; 27.0622x over previous
import functools

import jax
import jax.numpy as jnp
from jax import lax
from jax.experimental import pallas as pl
from jax.experimental.pallas import tpu as pltpu
from jax.experimental.pallas import tpu_sc as plsc

_EPS = 0.1
_N, _C, _W = 8192, 32000, 5
_NC, _NS, _L = 2, 16, 16
_NW = _NC * _NS
_RPW = _N // _NW
_TGT = _RPW
_WIN = _RPW * _W
_TOT = _TGT + _WIN
_CH = 128
_NCH = _TOT // _CH


def _loss_body(pred_hbm, tgt_hbm, smt_hbm, out_hbm,
               tgt_v, sm_v, idx_v, val_v, part_v, sem):
    cid = lax.axis_index("c")
    sid = lax.axis_index("s")
    wid = sid * _NC + cid
    base = wid * _RPW

    pltpu.sync_copy(tgt_hbm.at[pl.ds(base, _RPW)], tgt_v)
    for w in range(_W):
        pltpu.sync_copy(smt_hbm.at[pl.ds(w * _N + base, _RPW)],
                        sm_v.at[pl.ds(w * _RPW, _RPW)])

    lanes = lax.iota(jnp.int32, _L)

    def tiled_addr(rows, cols):
        return ((rows >> 3) * ((_C // 128) * 1024) + (cols >> 7) * 1024
                + (rows & 7) * 128 + (cols & 127))

    for k in range(_RPW // _L):
        rows = base + k * _L + lanes
        idx_v[pl.ds(k * _L, _L)] = tiled_addr(rows, tgt_v[pl.ds(k * _L, _L)])
    for w in range(_W):
        for k in range(_RPW // _L):
            rows = base + k * _L + lanes
            off = _TGT + w * _RPW + k * _L
            idx_v[pl.ds(off, _L)] = tiled_addr(
                rows, sm_v[pl.ds(w * _RPW + k * _L, _L)])

    copies = [
        pltpu.async_copy(pred_hbm.at[idx_v.at[pl.ds(c * _CH, _CH)]],
                         val_v.at[pl.ds(c * _CH, _CH)], sem)
        for c in range(_NCH)
    ]
    for cp in copies:
        cp.wait()

    acc_t = jnp.zeros((_L,), jnp.float32)
    for k in range(_TGT // _L):
        acc_t = acc_t + val_v[pl.ds(k * _L, _L)]
    acc_w = jnp.zeros((_L,), jnp.float32)
    for k in range(_WIN // _L):
        acc_w = acc_w + val_v[pl.ds(_TGT + k * _L, _L)]
    part_v[...] = -(acc_t * ((1.0 - _EPS) / _N) + acc_w * (_EPS / (_N * _W)))
    pltpu.sync_copy(part_v, out_hbm.at[wid])


@functools.partial(
    pl.kernel,
    out_type=jax.ShapeDtypeStruct((_NW, _L), jnp.float32),
    mesh=plsc.VectorSubcoreMesh(core_axis_name="c", subcore_axis_name="s"),
    scratch_types=[
        pltpu.VMEM((_TGT,), jnp.int32),
        pltpu.VMEM((_WIN,), jnp.int32),
        pltpu.VMEM((_TOT,), jnp.int32),
        pltpu.VMEM((_TOT,), jnp.float32),
        pltpu.VMEM((_L,), jnp.float32),
        pltpu.SemaphoreType.DMA,
    ],
)
def _sc_loss(pred_hbm, tgt_hbm, smt_hbm, out_hbm,
             tgt_v, sm_v, idx_v, val_v, part_v, sem):
    _loss_body(pred_hbm, tgt_hbm, smt_hbm, out_hbm,
               tgt_v, sm_v, idx_v, val_v, part_v, sem)


def kernel(pred, target, smooth_idx):
    pred_flat = (pred.reshape(_N // 8, 8, _C // 128, 128)
                 .transpose(0, 2, 1, 3)
                 .reshape(-1))
    tgt = target.astype(jnp.int32)
    smt = smooth_idx.astype(jnp.int32).T.reshape(-1)
    parts = _sc_loss(pred_flat, tgt, smt)
    return jnp.sum(parts)


# speedup vs baseline: 29.2414x; 1.0805x over previous
import functools

import jax
import jax.numpy as jnp
from jax import lax
from jax.experimental import pallas as pl
from jax.experimental.pallas import tpu as pltpu
from jax.experimental.pallas import tpu_sc as plsc

_EPS = 0.1
_N, _C, _W = 8192, 32000, 5
_NC, _NS, _L = 2, 16, 16
_NW = _NC * _NS
_RPW = _N // _NW
_TGT = _RPW
_WIN = _RPW * _W
_TOT = _TGT + _WIN
_CH = 128
_CPG = _RPW // _CH
_VPC = _CH // _L
_TPC = 1024
_CT = _C // 128


def _loss_body(pred_hbm, tgt_hbm, smt_hbm, out_hbm,
               tgt_v, sm_v, idx_v, val_v, part_v, sem_i, sem_g):
    cid = lax.axis_index("c")
    sid = lax.axis_index("s")
    wid = sid * _NC + cid
    base = wid * _RPW

    cp_t = pltpu.async_copy(tgt_hbm.at[pl.ds(base, _RPW)], tgt_v, sem_i)
    cp_s = [
        pltpu.async_copy(smt_hbm.at[pl.ds(w * _N + base, _RPW)],
                         sm_v.at[pl.ds(w * _RPW, _RPW)], sem_i)
        for w in range(_W)
    ]

    lanes = lax.iota(jnp.int32, _L)
    row_part = (base >> 3) * (_CT * _TPC)

    def emit_group(col_ref, col_off, idx_off, gathers):
        for k in range(_RPW // _L):
            o = k * _L + lanes
            rvec = (o >> 3) * (_CT * _TPC) + (o & 7) * 128 + row_part
            cols = col_ref[pl.ds(col_off + k * _L, _L)]
            ci = cols & 127
            idx_v[pl.ds(idx_off + k * _L, _L)] = rvec + ((cols - ci) << 3) + ci
            if (k + 1) % _VPC == 0:
                c0 = idx_off + (k + 1 - _VPC) * _L
                gathers.append(pltpu.async_copy(
                    pred_hbm.at[idx_v.at[pl.ds(c0, _CH)]],
                    val_v.at[pl.ds(c0, _CH)], sem_g))

    gathers = []
    cp_t.wait()
    emit_group(tgt_v, 0, 0, gathers)
    for cp in cp_s:
        cp.wait()
    for w in range(_W):
        emit_group(sm_v, w * _RPW, _TGT + w * _RPW, gathers)

    for cp in gathers:
        cp.wait()

    acc_t = jnp.zeros((_L,), jnp.float32)
    for k in range(_TGT // _L):
        acc_t = acc_t + val_v[pl.ds(k * _L, _L)]
    acc_w = jnp.zeros((_L,), jnp.float32)
    for k in range(_WIN // _L):
        acc_w = acc_w + val_v[pl.ds(_TGT + k * _L, _L)]
    part_v[...] = -(acc_t * ((1.0 - _EPS) / _N) + acc_w * (_EPS / (_N * _W)))
    pltpu.sync_copy(part_v, out_hbm.at[wid])


@functools.partial(
    pl.kernel,
    out_type=jax.ShapeDtypeStruct((_NW, _L), jnp.float32),
    mesh=plsc.VectorSubcoreMesh(core_axis_name="c", subcore_axis_name="s"),
    scratch_types=[
        pltpu.VMEM((_TGT,), jnp.int32),
        pltpu.VMEM((_WIN,), jnp.int32),
        pltpu.VMEM((_TOT,), jnp.int32),
        pltpu.VMEM((_TOT,), jnp.float32),
        pltpu.VMEM((_L,), jnp.float32),
        pltpu.SemaphoreType.DMA,
        pltpu.SemaphoreType.DMA,
    ],
)
def _sc_loss(pred_hbm, tgt_hbm, smt_hbm, out_hbm,
             tgt_v, sm_v, idx_v, val_v, part_v, sem_i, sem_g):
    _loss_body(pred_hbm, tgt_hbm, smt_hbm, out_hbm,
               tgt_v, sm_v, idx_v, val_v, part_v, sem_i, sem_g)


def kernel(pred, target, smooth_idx):
    pred_flat = (pred.reshape(_N // 8, 8, _C // 128, 128)
                 .transpose(0, 2, 1, 3)
                 .reshape(-1))
    tgt = target.astype(jnp.int32)
    smt = smooth_idx.astype(jnp.int32).T.reshape(-1)
    parts = _sc_loss(pred_flat, tgt, smt)
    return jnp.sum(parts)


# speedup vs baseline: 29.8368x; 1.0204x over previous
import functools

import jax
import jax.numpy as jnp
from jax import lax
from jax.experimental import pallas as pl
from jax.experimental.pallas import tpu as pltpu
from jax.experimental.pallas import tpu_sc as plsc

_EPS = 0.1
_N, _C, _W = 8192, 32000, 5
_NC, _NS, _L = 2, 16, 16
_NW = _NC * _NS
_RPW = _N // _NW
_TGT = _RPW
_WIN = _RPW * _W
_TOT = _TGT + _WIN
_CH = 128
_CPG = _RPW // _CH
_VPC = _CH // _L
_TPC = 1024
_CT = _C // 128


def _loss_body(pred_hbm, tgt_hbm, smt_hbm, out_hbm,
               tgt_v, sm_v, idx_v, val_v, part_v, sem_i, sem_g):
    cid = lax.axis_index("c")
    sid = lax.axis_index("s")
    wid = sid * _NC + cid
    base = wid * _RPW

    cp_t = pltpu.async_copy(tgt_hbm.at[pl.ds(base, _RPW)], tgt_v, sem_i)
    cp_s = [
        pltpu.async_copy(smt_hbm.at[pl.ds(w * _N + base, _RPW)],
                         sm_v.at[pl.ds(w * _RPW, _RPW)], sem_i)
        for w in range(_W)
    ]

    lanes = lax.iota(jnp.int32, _L)
    row_part = (base >> 3) * (_CT * _TPC)

    def emit_group(col_ref, col_off, idx_off, gathers):
        for k in range(_RPW // _L):
            o = k * _L + lanes
            rvec = (o >> 3) * (_CT * _TPC) + (o & 7) * 128 + row_part
            cols = col_ref[pl.ds(col_off + k * _L, _L)]
            ci = cols & 127
            idx_v[pl.ds(idx_off + k * _L, _L)] = rvec + ((cols - ci) << 3) + ci
            if (k + 1) % _VPC == 0:
                c0 = idx_off + (k + 1 - _VPC) * _L
                gathers.append(pltpu.async_copy(
                    pred_hbm.at[idx_v.at[pl.ds(c0, _CH)]],
                    val_v.at[pl.ds(c0, _CH)], sem_g.at[len(gathers)]))

    gathers = []
    cp_t.wait()
    emit_group(tgt_v, 0, 0, gathers)
    for cp in cp_s:
        cp.wait()
    for w in range(_W):
        emit_group(sm_v, w * _RPW, _TGT + w * _RPW, gathers)

    acc_t = jnp.zeros((_L,), jnp.float32)
    acc_w = jnp.zeros((_L,), jnp.float32)
    for c in range(len(gathers)):
        gathers[c].wait()
        for k in range(_VPC):
            v = val_v[pl.ds(c * _CH + k * _L, _L)]
            if c < _TGT // _CH:
                acc_t = acc_t + v
            else:
                acc_w = acc_w + v
    part_v[...] = -(acc_t * ((1.0 - _EPS) / _N) + acc_w * (_EPS / (_N * _W)))
    pltpu.sync_copy(part_v, out_hbm.at[wid])


@functools.partial(
    pl.kernel,
    out_type=jax.ShapeDtypeStruct((_NW, _L), jnp.float32),
    mesh=plsc.VectorSubcoreMesh(core_axis_name="c", subcore_axis_name="s"),
    scratch_types=[
        pltpu.VMEM((_TGT,), jnp.int32),
        pltpu.VMEM((_WIN,), jnp.int32),
        pltpu.VMEM((_TOT,), jnp.int32),
        pltpu.VMEM((_TOT,), jnp.float32),
        pltpu.VMEM((_L,), jnp.float32),
        pltpu.SemaphoreType.DMA,
        pltpu.SemaphoreType.DMA((_TOT // _CH,)),
    ],
)
def _sc_loss(pred_hbm, tgt_hbm, smt_hbm, out_hbm,
             tgt_v, sm_v, idx_v, val_v, part_v, sem_i, sem_g):
    _loss_body(pred_hbm, tgt_hbm, smt_hbm, out_hbm,
               tgt_v, sm_v, idx_v, val_v, part_v, sem_i, sem_g)


def kernel(pred, target, smooth_idx):
    pred_flat = (pred.reshape(_N // 8, 8, _C // 128, 128)
                 .transpose(0, 2, 1, 3)
                 .reshape(-1))
    tgt = target.astype(jnp.int32)
    smt = smooth_idx.astype(jnp.int32).T.reshape(-1)
    parts = _sc_loss(pred_flat, tgt, smt)
    return jnp.sum(parts)


# speedup vs baseline: 30.4413x; 1.0203x over previous
import functools

import jax
import jax.numpy as jnp
from jax import lax
from jax.experimental import pallas as pl
from jax.experimental.pallas import tpu as pltpu
from jax.experimental.pallas import tpu_sc as plsc

_EPS = 0.1
_N, _C, _W = 8192, 32000, 5
_NC, _NS, _L = 2, 16, 16
_NW = _NC * _NS
_RPW = _N // _NW
_TGT = _RPW
_WIN = _RPW * _W
_TOT = _TGT + _WIN
_CH = 128
_NCH = _TOT // _CH
_VPC = _CH // _L
_TPC = 1024
_ROWT = (_C // 128) * _TPC


def _loss_body(pred_hbm, tgt_hbm, smt_hbm, out_hbm,
               tgt_v, sm_v, idx_v, val_v, part_v, sem_i, sem_g):
    cid = lax.axis_index("c")
    sid = lax.axis_index("s")
    wid = sid * _NC + cid
    base = wid * _RPW

    cp_t = pltpu.async_copy(tgt_hbm.at[pl.ds(base, _RPW)], tgt_v, sem_i)
    cp_s = [
        pltpu.async_copy(smt_hbm.at[pl.ds(w * _N + base, _RPW)],
                         sm_v.at[pl.ds(w * _RPW, _RPW)], sem_i)
        for w in range(_W)
    ]

    lanes = lax.iota(jnp.int32, _L)
    row_part = (base >> 3) * _ROWT
    lane_rvec = (lanes >> 3) * _ROWT + (lanes & 7) * 128

    def fire(c):
        return pltpu.async_copy(
            pred_hbm.at[idx_v.at[pl.ds(c * _CH, _CH)]],
            val_v.at[pl.ds(c * _CH, _CH)], sem_g.at[c])

    cp_t.wait()
    for k in range(_TGT // _L):
        cols = tgt_v[pl.ds(k * _L, _L)]
        ci = cols & 127
        idx_v[pl.ds(k * _L, _L)] = (
            row_part + 2 * k * _ROWT + lane_rvec + ((cols - ci) << 3) + ci)
        if (k + 1) % _VPC == 0:
            fire((k + 1) // _VPC - 1)
    for cp in cp_s:
        cp.wait()

    def wbody(k, carry):
        j16 = k * _L
        s = lax.rem(j16, 5)
        q = lax.div(j16 - s, 5)
        sl = s + lanes
        p = (jnp.where(sl >= 5, 1, 0) + jnp.where(sl >= 10, 1, 0)
             + jnp.where(sl >= 15, 1, 0))
        r_off = q + p
        rvec = row_part + (r_off >> 3) * _ROWT + (r_off & 7) * 128
        cols = sm_v[pl.ds(j16, _L)]
        ci = cols & 127
        idx_v[pl.ds(_TGT + j16, _L)] = rvec + ((cols - ci) << 3) + ci

        @pl.when((k & (_VPC - 1)) == _VPC - 1)
        def _():
            c = _TGT // _CH + (k >> 3)
            pltpu.async_copy(
                pred_hbm.at[idx_v.at[pl.ds(c * _CH, _CH)]],
                val_v.at[pl.ds(c * _CH, _CH)], sem_g.at[c])

        return carry

    lax.fori_loop(0, _WIN // _L, wbody, 0)

    acc_t = jnp.zeros((_L,), jnp.float32)
    acc_w = jnp.zeros((_L,), jnp.float32)
    for c in range(_NCH):
        pltpu.make_async_copy(
            pred_hbm.at[idx_v.at[pl.ds(c * _CH, _CH)]],
            val_v.at[pl.ds(c * _CH, _CH)], sem_g.at[c]).wait()
        for k in range(_VPC):
            v = val_v[pl.ds(c * _CH + k * _L, _L)]
            if c < _TGT // _CH:
                acc_t = acc_t + v
            else:
                acc_w = acc_w + v
    part_v[...] = -(acc_t * ((1.0 - _EPS) / _N) + acc_w * (_EPS / (_N * _W)))
    pltpu.sync_copy(part_v, out_hbm.at[wid])


@functools.partial(
    pl.kernel,
    out_type=jax.ShapeDtypeStruct((_NW, _L), jnp.float32),
    mesh=plsc.VectorSubcoreMesh(core_axis_name="c", subcore_axis_name="s"),
    scratch_types=[
        pltpu.VMEM((_TGT,), jnp.int32),
        pltpu.VMEM((_WIN,), jnp.int32),
        pltpu.VMEM((_TOT,), jnp.int32),
        pltpu.VMEM((_TOT,), jnp.float32),
        pltpu.VMEM((_L,), jnp.float32),
        pltpu.SemaphoreType.DMA,
        pltpu.SemaphoreType.DMA((_NCH,)),
    ],
)
def _sc_loss(pred_hbm, tgt_hbm, smt_hbm, out_hbm,
             tgt_v, sm_v, idx_v, val_v, part_v, sem_i, sem_g):
    _loss_body(pred_hbm, tgt_hbm, smt_hbm, out_hbm,
               tgt_v, sm_v, idx_v, val_v, part_v, sem_i, sem_g)


def kernel(pred, target, smooth_idx):
    pred_flat = (pred.reshape(_N // 8, 8, _C // 128, 128)
                 .transpose(0, 2, 1, 3)
                 .reshape(-1))
    tgt = target.astype(jnp.int32)
    smt = smooth_idx.astype(jnp.int32).T.reshape(-1)
    parts = _sc_loss(pred_flat, tgt, smt)
    return jnp.sum(parts)
